# baseline (speedup 1.0000x reference)
import functools

import jax
import jax.numpy as jnp
from jax import lax
from jax.experimental import pallas as pl
from jax.experimental.pallas import tpu as pltpu
from jax.experimental.pallas import tpu_sc as plsc

B = 130000
V = 1000000
H = 64
L = 16
NC, NS = 2, 16
NW = NC * NS
BP = 131072
BW = BP // NW
CH = 128
NCHUNK = BW // CH
NB = 2
NGRP = NCHUNK // NB
RW = BW // CH

_MESH = dict(core_axis_name="c", subcore_axis_name="s")


def _sc_wgather(wrd2d, word_emb):
    @functools.partial(
        pl.kernel,
        compiler_params=pltpu.CompilerParams(
            needs_layout_passes=False, use_tc_tiling_on_sc=False),
        out_type=jax.ShapeDtypeStruct((BP, H), jnp.float32),
        mesh=plsc.VectorSubcoreMesh(**_MESH),
        scratch_types=[
            pltpu.VMEM((NCHUNK, CH), jnp.int32),
            [pltpu.VMEM((CH, H), jnp.float32)] * NB,
            [pltpu.SemaphoreType.DMA] * NB,
            [pltpu.SemaphoreType.DMA] * NB,
        ],
    )
    def k(wrd_h, wemb_h, out_h, widx, wrows, gsems, osems):
        wid = lax.axis_index("s") * NC + lax.axis_index("c")
        r0 = pl.multiple_of(wid * RW, RW)
        i0 = pl.multiple_of(wid * BW, BW)

        pltpu.sync_copy(wrd_h.at[pl.ds(r0, RW)], widx)

        def gather_start(g, b):
            pltpu.make_async_copy(
                wemb_h.at[widx.at[g]], wrows[b], gsems[b]).start()

        def gather_wait(g, b):
            pltpu.make_async_copy(
                wemb_h.at[widx.at[g]], wrows[b], gsems[b]).wait()

        def write_start(g, b):
            dst = out_h.at[pl.ds(i0 + g * CH, CH)]
            pltpu.make_async_copy(wrows[b], dst, osems[b]).start()

        def write_wait(b):
            pltpu.make_async_copy(
                wrows[b], out_h.at[pl.ds(0, CH)], osems[b]).wait()

        for b in range(NB):
            gather_start(b, b)

        def grp_body(grp, _):
            for b in range(NB):
                g = grp * NB + b
                gather_wait(g, b)

                @pl.when(grp > 0)
                def _():
                    write_wait(b)
                write_start(g, b)

                @pl.when(grp < NGRP - 1)
                def _():
                    gather_start(g + NB, b)
            return 0

        lax.fori_loop(0, NGRP, grp_body, 0)
        for b in range(NB):
            write_wait(b)

    return k(wrd2d, word_emb)


def _sc_cdots(cntxt2d, context_emb, wflat):
    @functools.partial(
        pl.kernel,
        compiler_params=pltpu.CompilerParams(
            needs_layout_passes=False, use_tc_tiling_on_sc=True),
        out_type=jax.ShapeDtypeStruct((BP // CH, CH), jnp.float32),
        mesh=plsc.VectorSubcoreMesh(**_MESH),
        scratch_types=[
            pltpu.VMEM((NCHUNK, CH), jnp.int32),
            [pltpu.VMEM((CH, H), jnp.float32)] * NB,
            [pltpu.VMEM((CH * H // 128, 128), jnp.float32)] * NB,
            pltpu.VMEM((NCHUNK, CH), jnp.float32),
            [pltpu.SemaphoreType.DMA] * NB,
            [pltpu.SemaphoreType.DMA] * NB,
        ],
    )
    def k(cx_h, cemb_h, wflat_h, out_h,
          cidx, crows, wbufs, dots, csems, wsems):
        wid = lax.axis_index("s") * NC + lax.axis_index("c")
        r0 = pl.multiple_of(wid * RW, RW)
        WROWS = CH * H // 128
        w0 = pl.multiple_of(wid * (BW * H // 128), BW * H // 128)

        pltpu.sync_copy(cx_h.at[pl.ds(r0, RW)], cidx)

        def chunk_start(g, b):
            pltpu.make_async_copy(
                wflat_h.at[pl.ds(w0 + g * WROWS, WROWS)],
                wbufs[b], wsems[b]).start()

            def sub(s, _):
                off = pl.multiple_of(s * L, L)
                ivc = cidx[g, pl.ds(off, L)]
                for j in range(L):
                    pltpu.make_async_copy(
                        cemb_h.at[pl.ds(ivc[j], 1)],
                        crows[b].at[pl.ds(s * L + j, 1)], csems[b]).start()
                return 0
            lax.fori_loop(0, CH // L, sub, 0)

        def chunk_wait(b):
            pltpu.make_async_copy(
                wflat_h.at[pl.ds(0, WROWS)], wbufs[b], wsems[b]).wait()
            pltpu.make_async_copy(
                cemb_h.at[pl.ds(0, CH)], crows[b], csems[b]).wait()

        for b in range(NB):
            chunk_start(b, b)

        lanes = lax.iota(jnp.int32, L)

        par64 = (lanes & 1) * H

        def dotgroup(wb, cr, iv, rv):
            accs = [jnp.zeros((L,), jnp.float32) for _ in range(4)]
            for h in range(H):
                hv = jnp.full((L,), h, jnp.int32)
                wv = plsc.load_gather(wb, [rv, par64 + h])
                cv = plsc.load_gather(cr, [iv, hv])
                accs[h % 4] = accs[h % 4] + wv * cv
            return (accs[0] + accs[1]) + (accs[2] + accs[3])

        def compute(g, b):
            def body(j, _):
                iv = jnp.full((L,), j * L, jnp.int32) + lanes
                rv = jnp.full((L,), j * (L // 2), jnp.int32) + (lanes >> 1)
                off = pl.multiple_of(j * L, L)
                dots[g, pl.ds(off, L)] = dotgroup(
                    wbufs[b], crows[b], iv, rv)
                return 0
            lax.fori_loop(0, CH // L, body, 0)

        def grp_body(grp, _):
            for b in range(NB):
                g = grp * NB + b
                chunk_wait(b)
                compute(g, b)

                @pl.when(grp < NGRP - 1)
                def _():
                    chunk_start(g + NB, b)
            return 0

        lax.fori_loop(0, NGRP, grp_body, 0)

        pltpu.sync_copy(dots, out_h.at[pl.ds(r0, RW)])

    return k(cntxt2d, context_emb, wflat)


def _tc_loss(dots2d, labels2d):
    def body(d_ref, l_ref, o_ref):
        x = d_ref[...] * l_ref[...]
        r = lax.broadcasted_iota(jnp.int32, x.shape, 0)
        c = lax.broadcasted_iota(jnp.int32, x.shape, 1)
        valid = (r * x.shape[1] + c) < B
        ls = jnp.where(valid, jax.nn.log_sigmoid(x), 0.0)
        o_ref[0, 0] = jnp.sum(ls) * (-1.0 / B)

    out = pl.pallas_call(
        body,
        out_shape=jax.ShapeDtypeStruct((1, 1), jnp.float32),
        out_specs=pl.BlockSpec(memory_space=pltpu.SMEM),
    )(dots2d, labels2d)
    return out[0, 0]


def kernel(wrd, cntxt, labels, word_emb, context_emb):
    pad = BP - B
    wrd_p = jnp.concatenate(
        [wrd.reshape(-1), jnp.zeros((pad,), jnp.int32)]).reshape(BP // CH, CH)
    cx_p = jnp.concatenate(
        [cntxt.reshape(-1), jnp.zeros((pad,), jnp.int32)]).reshape(BP // CH, CH)
    lab_p = jnp.concatenate(
        [labels.reshape(-1), jnp.zeros((pad,), jnp.float32)]).reshape(BP // CH, CH)
    wrows = _sc_wgather(wrd_p, word_emb)
    dots = _sc_cdots(cx_p, context_emb, wrows.reshape(BP * H // 128, 128))
    return _tc_loss(dots, lab_p)
